# baseline (speedup 1.0000x reference)
import functools

import jax
import jax.numpy as jnp
from jax import lax
from jax.experimental import pallas as pl
from jax.experimental.pallas import tpu as pltpu
from jax.experimental.pallas import tpu_sc as plsc

_HID = 768
_NSL = _HID // 16
_G = 16
_EPS = 1e-12


def _dyn_gather(v, idx):
  return lax.gather(
      v, idx.reshape(16, 1),
      dimension_numbers=lax.GatherDimensionNumbers(
          offset_dims=(), collapsed_slice_dims=(0,), start_index_map=(0,)),
      slice_sizes=(1,),
      mode=lax.GatherScatterMode.PROMISE_IN_BOUNDS)


def _allsum(v):
  lanes = lax.iota(jnp.int32, 16)
  for k in (1, 2, 4, 8):
    v = v + _dyn_gather(v, lanes ^ k)
  return v


def _rsqrt(var):
  iv = lax.bitcast_convert_type(var, jnp.int32)
  y = lax.bitcast_convert_type(
      jnp.int32(0x5F3759DF) - lax.shift_right_logical(iv, 1), jnp.float32)
  for _ in range(3):
    y = y * (1.5 - 0.5 * var * y * y)
  return y


def _make_sc_kernel(n_tok, seq_len):
  info = plsc.get_sparse_core_info()
  nc, ns = info.num_cores, info.num_subcores
  nw = nc * ns
  tpw = n_tok // nw
  nseq = tpw // seq_len
  npch = seq_len // _G

  mesh = plsc.VectorSubcoreMesh(core_axis_name="c", subcore_axis_name="s")

  @functools.partial(
      pl.kernel,
      mesh=mesh,
      out_type=jax.ShapeDtypeStruct((n_tok, _HID), jnp.float32),
      scratch_types=[
          pltpu.VMEM((tpw,), jnp.int32),
          pltpu.VMEM((tpw,), jnp.int32),
          pltpu.VMEM((_G,), jnp.int32),
          pltpu.VMEM((_G,), jnp.int32),
          pltpu.VMEM((_G,), jnp.int32),
          pltpu.VMEM((_G,), jnp.int32),
          pltpu.VMEM((_G, _HID), jnp.float32),
          pltpu.VMEM((_G, _HID), jnp.float32),
          pltpu.VMEM((_G, _HID), jnp.float32),
          pltpu.VMEM((_G, _HID), jnp.float32),
          pltpu.VMEM((_G, _HID), jnp.float32),
          pltpu.VMEM((2, _HID), jnp.float32),
          pltpu.VMEM((_HID,), jnp.float32),
          pltpu.VMEM((_HID,), jnp.float32),
          pltpu.VMEM((_HID,), jnp.float32),
          pltpu.SemaphoreType.DMA,
          pltpu.SemaphoreType.DMA,
          pltpu.SemaphoreType.DMA,
          pltpu.SemaphoreType.DMA,
          pltpu.SemaphoreType.DMA,
      ],
  )
  def sc_kernel(ids_h, tt_h, ww_h, wp_h, wt_h, g_h, b_h, out_h,
                ids_v, tt_v, idxb0, idxb1, idxb2, idxb3,
                rows0, rows1, rows2, rows3, pos_v, type_v,
                tdiff_v, gamma_v, beta_v, gsem, ssem0, ssem1, ssem2, ssem3):
    rows = (rows0, rows1, rows2, rows3)
    idxb = (idxb0, idxb1, idxb2, idxb3)
    ssem = (ssem0, ssem1, ssem2, ssem3)
    wid = lax.axis_index("s") * nc + lax.axis_index("c")
    base = wid * tpw
    pltpu.sync_copy(ids_h.at[pl.ds(base, tpw)], ids_v)
    pltpu.sync_copy(tt_h.at[pl.ds(base, tpw)], tt_v)
    pltpu.sync_copy(wt_h, type_v)
    pltpu.sync_copy(g_h, gamma_v)
    pltpu.sync_copy(b_h, beta_v)

    def mk_tdiff(i, c):
      sl = pl.ds(i * 16, 16)
      tdiff_v[sl] = type_v[1, sl] - type_v[0, sl]
      return c
    lax.fori_loop(0, _NSL, mk_tdiff, 0)

    def compute(t, buf):
      ttf = tt_v[pl.ds(t, _G)].astype(jnp.float32)
      tsel = [_dyn_gather(ttf, jnp.full((16,), j, jnp.int32))
              for j in range(_G)]

      def a12(i, c):
        a, a2 = c
        sl = pl.ds(i * 16, 16)
        td = tdiff_v[sl]
        na, na2 = [], []
        for j in range(_G):
          x = buf[j, sl] + pos_v[j, sl] + tsel[j] * td
          buf[j, sl] = x
          na.append(a[j] + x)
          na2.append(a2[j] + x * x)
        return (tuple(na), tuple(na2))
      zeros = tuple(jnp.zeros((16,), jnp.float32) for _ in range(_G))
      accs, accs2 = plsc.parallel_loop(
          0, _NSL, unroll=2, carry=(zeros, zeros))(a12)

      inv = []
      m2 = []
      cinv = 1.0 / _HID
      for j in range(_G):
        mean = _allsum(accs[j]) * cinv
        var = _allsum(accs2[j]) * cinv - mean * mean + _EPS
        y = _rsqrt(var)
        inv.append(y)
        m2.append(mean * y)

      @plsc.parallel_loop(0, _NSL, unroll=2)
      def c1(i):
        sl = pl.ds(i * 16, 16)
        g = gamma_v[sl]
        bt = beta_v[sl]
        for j in range(_G):
          buf[j, sl] = (buf[j, sl] * inv[j] - m2[j]) * g + bt

    idxb0[...] = ids_v[pl.ds(0, _G)]
    pltpu.async_copy(ww_h.at[idxb0], rows0, gsem)
    last = base + (npch - 1) * _G + (nseq - 1) * seq_len
    pltpu.async_copy(pos_v, out_h.at[pl.ds(last, _G)], ssem1)
    pltpu.async_copy(pos_v, out_h.at[pl.ds(last, _G)], ssem2)
    pltpu.async_copy(pos_v, out_h.at[pl.ds(last, _G)], ssem3)

    def outer(kk, c):
      pbase = kk * _G
      pltpu.sync_copy(wp_h.at[pl.ds(pbase, _G)], pos_v)

      @plsc.parallel_loop(0, _NSL, unroll=2)
      def fold(i):
        sl = pl.ds(i * 16, 16)
        t0 = type_v[0, sl]
        for p in range(_G):
          pos_v[p, sl] = pos_v[p, sl] + t0

      for s in range(nseq):
        b = s
        nb = (s + 1) % nseq
        buf = rows[b]
        nbuf = rows[nb]
        toff = pbase + s * seq_len

        pltpu.make_async_copy(ww_h.at[idxb[b]], buf, gsem).wait()
        pltpu.make_async_copy(nbuf, out_h.at[pl.ds(base, _G)],
                              ssem[nb]).wait()
        toff1 = pbase + (s + 1) * seq_len if s < nseq - 1 else pbase + _G
        idxb[nb][...] = ids_v[pl.ds(toff1, _G)]
        pltpu.async_copy(ww_h.at[idxb[nb]], nbuf, gsem)

        compute(toff, buf)
        pltpu.async_copy(buf, out_h.at[pl.ds(base + toff, _G)], ssem[b])
      return c
    lax.fori_loop(0, npch, outer, 0)

    pltpu.make_async_copy(ww_h.at[idxb0], rows0, gsem).wait()
    pltpu.make_async_copy(rows1, out_h.at[pl.ds(base, _G)], ssem1).wait()
    pltpu.make_async_copy(rows2, out_h.at[pl.ds(base, _G)], ssem2).wait()
    pltpu.make_async_copy(rows3, out_h.at[pl.ds(base, _G)], ssem3).wait()

  return sc_kernel


def kernel(input_ids, token_type_ids, W_word, W_pos, W_type, gamma, beta):
  b, l = input_ids.shape
  n_tok = b * l
  ids_flat = input_ids.reshape(n_tok).astype(jnp.int32)
  tt_flat = token_type_ids.reshape(n_tok).astype(jnp.int32)
  sc = _make_sc_kernel(n_tok, l)
  out = sc(ids_flat, tt_flat, W_word, W_pos, W_type, gamma, beta)
  return out.reshape(b, l, _HID)


# speedup vs baseline: 1.0000x; 1.0000x over previous
import functools

import jax
import jax.numpy as jnp
from jax import lax
from jax.experimental import pallas as pl
from jax.experimental.pallas import tpu as pltpu
from jax.experimental.pallas import tpu_sc as plsc

_HID = 768
_NSL = _HID // 16
_G = 16
_EPS = 1e-12


def _dyn_gather(v, idx):
  return lax.gather(
      v, idx.reshape(16, 1),
      dimension_numbers=lax.GatherDimensionNumbers(
          offset_dims=(), collapsed_slice_dims=(0,), start_index_map=(0,)),
      slice_sizes=(1,),
      mode=lax.GatherScatterMode.PROMISE_IN_BOUNDS)


def _allsum(v):
  lanes = lax.iota(jnp.int32, 16)
  for k in (1, 2, 4, 8):
    v = v + _dyn_gather(v, lanes ^ k)
  return v


def _rsqrt(var):
  iv = lax.bitcast_convert_type(var, jnp.int32)
  y = lax.bitcast_convert_type(
      jnp.int32(0x5F3759DF) - lax.shift_right_logical(iv, 1), jnp.float32)
  for _ in range(3):
    y = y * (1.5 - 0.5 * var * y * y)
  return y


def _make_sc_kernel(n_tok, seq_len):
  info = plsc.get_sparse_core_info()
  nc, ns = info.num_cores, info.num_subcores
  nw = nc * ns
  tpw = n_tok // nw
  nseq = tpw // seq_len
  npch = seq_len // _G

  mesh = plsc.VectorSubcoreMesh(core_axis_name="c", subcore_axis_name="s")

  @functools.partial(
      pl.kernel,
      mesh=mesh,
      out_type=jax.ShapeDtypeStruct((n_tok, _HID), jnp.float32),
      scratch_types=[
          pltpu.VMEM((tpw,), jnp.int32),
          pltpu.VMEM((tpw,), jnp.int32),
          pltpu.VMEM((_G,), jnp.int32),
          pltpu.VMEM((_G,), jnp.int32),
          pltpu.VMEM((_G,), jnp.int32),
          pltpu.VMEM((_G,), jnp.int32),
          pltpu.VMEM((_G, _HID), jnp.float32),
          pltpu.VMEM((_G, _HID), jnp.float32),
          pltpu.VMEM((_G, _HID), jnp.float32),
          pltpu.VMEM((_G, _HID), jnp.float32),
          pltpu.VMEM((_G, _HID), jnp.float32),
          pltpu.VMEM((2, _HID), jnp.float32),
          pltpu.VMEM((_HID,), jnp.float32),
          pltpu.VMEM((_HID,), jnp.float32),
          pltpu.VMEM((_HID,), jnp.float32),
          pltpu.SemaphoreType.DMA,
          pltpu.SemaphoreType.DMA,
          pltpu.SemaphoreType.DMA,
          pltpu.SemaphoreType.DMA,
          pltpu.SemaphoreType.DMA,
      ],
  )
  def sc_kernel(ids_h, tt_h, ww_h, wp_h, wt_h, g_h, b_h, out_h,
                ids_v, tt_v, idxb0, idxb1, idxb2, idxb3,
                rows0, rows1, rows2, rows3, pos_v, type_v,
                tdiff_v, gamma_v, beta_v, gsem, ssem0, ssem1, ssem2, ssem3):
    rows = (rows0, rows1, rows2, rows3)
    idxb = (idxb0, idxb1, idxb2, idxb3)
    ssem = (ssem0, ssem1, ssem2, ssem3)
    wid = lax.axis_index("s") * nc + lax.axis_index("c")
    base = wid * tpw
    pltpu.sync_copy(ids_h.at[pl.ds(base, tpw)], ids_v)
    pltpu.sync_copy(tt_h.at[pl.ds(base, tpw)], tt_v)
    pltpu.sync_copy(wt_h, type_v)
    pltpu.sync_copy(g_h, gamma_v)
    pltpu.sync_copy(b_h, beta_v)

    def mk_tdiff(i, c):
      sl = pl.ds(i * 16, 16)
      tdiff_v[sl] = type_v[1, sl] - type_v[0, sl]
      return c
    lax.fori_loop(0, _NSL, mk_tdiff, 0)

    def compute(t, buf):
      ttf = tt_v[pl.ds(t, _G)].astype(jnp.float32)
      tsel = [_dyn_gather(ttf, jnp.full((16,), j, jnp.int32))
              for j in range(_G)]

      def a12(i, c):
        a, a2 = c
        sl = pl.ds(i * 16, 16)
        td = tdiff_v[sl]
        t0 = type_v[0, sl]
        na, na2 = [], []
        for j in range(_G):
          x = buf[j, sl] + t0 + pos_v[j, sl] + tsel[j] * td
          buf[j, sl] = x
          na.append(a[j] + x)
          na2.append(a2[j] + x * x)
        return (tuple(na), tuple(na2))
      zeros = tuple(jnp.zeros((16,), jnp.float32) for _ in range(_G))
      accs, accs2 = plsc.parallel_loop(
          0, _NSL, unroll=2, carry=(zeros, zeros))(a12)

      inv = []
      m2 = []
      cinv = 1.0 / _HID
      for j in range(_G):
        mean = _allsum(accs[j]) * cinv
        var = _allsum(accs2[j]) * cinv - mean * mean + _EPS
        y = _rsqrt(var)
        inv.append(y)
        m2.append(mean * y)

      @plsc.parallel_loop(0, _NSL, unroll=2)
      def c1(i):
        sl = pl.ds(i * 16, 16)
        g = gamma_v[sl]
        bt = beta_v[sl]
        for j in range(_G):
          buf[j, sl] = (buf[j, sl] * inv[j] - m2[j]) * g + bt

    idxb0[...] = ids_v[pl.ds(0, _G)]
    pltpu.async_copy(ww_h.at[idxb0], rows0, gsem)
    idxb1[...] = ids_v[pl.ds(seq_len, _G)]
    pltpu.async_copy(ww_h.at[idxb1], rows1, gsem)
    last = base + (npch - 1) * _G + (nseq - 1) * seq_len
    pltpu.async_copy(pos_v, out_h.at[pl.ds(last, _G)], ssem2)
    pltpu.async_copy(pos_v, out_h.at[pl.ds(last, _G)], ssem3)

    def outer(kk, c):
      pbase = kk * _G
      pltpu.sync_copy(wp_h.at[pl.ds(pbase, _G)], pos_v)

      for s in range(nseq):
        b = s
        nb = (s + 2) % nseq
        buf = rows[b]
        nbuf = rows[nb]
        toff = pbase + s * seq_len

        pltpu.make_async_copy(ww_h.at[idxb[b]], buf, gsem).wait()
        pltpu.make_async_copy(nbuf, out_h.at[pl.ds(base, _G)],
                              ssem[nb]).wait()
        toff2 = (pbase + (s + 2) * seq_len if s < nseq - 2
                 else pbase + _G + (s - 2) * seq_len)
        idxb[nb][...] = ids_v[pl.ds(toff2, _G)]
        pltpu.async_copy(ww_h.at[idxb[nb]], nbuf, gsem)

        compute(toff, buf)
        pltpu.async_copy(buf, out_h.at[pl.ds(base + toff, _G)], ssem[b])
      return c
    lax.fori_loop(0, npch, outer, 0)

    pltpu.make_async_copy(ww_h.at[idxb0], rows0, gsem).wait()
    pltpu.make_async_copy(ww_h.at[idxb1], rows1, gsem).wait()
    pltpu.make_async_copy(rows2, out_h.at[pl.ds(base, _G)], ssem2).wait()
    pltpu.make_async_copy(rows3, out_h.at[pl.ds(base, _G)], ssem3).wait()

  return sc_kernel


def kernel(input_ids, token_type_ids, W_word, W_pos, W_type, gamma, beta):
  b, l = input_ids.shape
  n_tok = b * l
  ids_flat = input_ids.reshape(n_tok).astype(jnp.int32)
  tt_flat = token_type_ids.reshape(n_tok).astype(jnp.int32)
  sc = _make_sc_kernel(n_tok, l)
  out = sc(ids_flat, tt_flat, W_word, W_pos, W_type, gamma, beta)
  return out.reshape(b, l, _HID)
